# baseline (speedup 1.0000x reference)
import jax
import jax.numpy as jnp
from jax.experimental import pallas as pl

_NEG_SLOPE = 0.1
_LANES = 128
_BLOCK_ROWS = 1000


def _prelu_body(src_ref, snk_ref, out_src_ref, out_snk_ref):
    x = src_ref[...]
    out_src_ref[...] = jnp.where(x >= 0, x, _NEG_SLOPE * x)
    y = snk_ref[...]
    out_snk_ref[...] = jnp.where(y >= 0, y, _NEG_SLOPE * y)


def kernel(source_emb, sink_emb, source_edge_index, sink_edge_index,
           W_src, a_src_s, a_src_d, b_src,
           W_snk, a_snk_s, a_snk_d, b_snk):
    n, d = source_emb.shape
    rows = n * d // _LANES
    src = source_emb.reshape(rows, _LANES)
    snk = sink_emb.reshape(rows, _LANES)

    block = (_BLOCK_ROWS, _LANES)
    grid = (rows // _BLOCK_ROWS,)
    spec = pl.BlockSpec(block, lambda i: (i, 0))
    out_src, out_snk = pl.pallas_call(
        _prelu_body,
        grid=grid,
        in_specs=[spec, spec],
        out_specs=[spec, spec],
        out_shape=[
            jax.ShapeDtypeStruct((rows, _LANES), source_emb.dtype),
            jax.ShapeDtypeStruct((rows, _LANES), sink_emb.dtype),
        ],
    )(src, snk)
    return (out_src.reshape(n, d), out_snk.reshape(n, d))


# speedup vs baseline: 1.2075x; 1.2075x over previous
import jax
import jax.numpy as jnp
from jax.experimental import pallas as pl

_NEG_SLOPE = 0.1
_BLOCK_ROWS = 4000


def _prelu_body(src_ref, snk_ref, out_src_ref, out_snk_ref):
    x = src_ref[...]
    out_src_ref[...] = jnp.where(x >= 0, x, _NEG_SLOPE * x)
    y = snk_ref[...]
    out_snk_ref[...] = jnp.where(y >= 0, y, _NEG_SLOPE * y)


def kernel(source_emb, sink_emb, source_edge_index, sink_edge_index,
           W_src, a_src_s, a_src_d, b_src,
           W_snk, a_snk_s, a_snk_d, b_snk):
    n, d = source_emb.shape
    block = (_BLOCK_ROWS, d)
    grid = (n // _BLOCK_ROWS,)
    spec = pl.BlockSpec(block, lambda i: (i, 0))
    out_src, out_snk = pl.pallas_call(
        _prelu_body,
        grid=grid,
        in_specs=[spec, spec],
        out_specs=[spec, spec],
        out_shape=[
            jax.ShapeDtypeStruct((n, d), source_emb.dtype),
            jax.ShapeDtypeStruct((n, d), sink_emb.dtype),
        ],
    )(source_emb, sink_emb)
    return (out_src, out_snk)
